# Initial kernel scaffold; baseline (speedup 1.0000x reference)
import functools

import jax
import jax.numpy as jnp
from jax import lax
from jax.experimental import pallas as pl
from jax.experimental.pallas import tpu as pltpu
from jax.experimental.pallas import tpu_sc as plsc

N = 50000
E = 800000
G = 128
D_IN = 64
D_HID = 16
D_OUT = 10

NC = 2
NS = 16
CHUNK = 128
EDGES_PER_CORE = E // NC
CHUNKS_PER_CORE = EDGES_PER_CORE // CHUNK
CHUNK_ITERS = (CHUNKS_PER_CORE + NS - 1) // NS
ROWS_PER_TILE = N // NS

BLK = 2000
NB = N // BLK


def _sc_mesh():
    return plsc.VectorSubcoreMesh(core_axis_name="c", subcore_axis_name="s")


def _sc_agg(g, src, dst, zeros):
    @functools.partial(
        pl.kernel,
        out_type=jax.ShapeDtypeStruct((NC, N, D_HID), jnp.float32),
        mesh=_sc_mesh(),
        scratch_types=[
            pltpu.VMEM((CHUNK,), jnp.int32),
            pltpu.VMEM((CHUNK,), jnp.int32),
            pltpu.VMEM((CHUNK, D_HID), jnp.float32),
            pltpu.VMEM_SHARED((N, D_HID), jnp.float32),
        ],
    )
    def body(g_hbm, src_hbm, dst_hbm, z_hbm, out_hbm, sidx, didx, rows, acc):
        c = lax.axis_index("c")
        s = lax.axis_index("s")
        row0 = s * ROWS_PER_TILE
        pltpu.sync_copy(z_hbm.at[pl.ds(row0, ROWS_PER_TILE)],
                        acc.at[pl.ds(row0, ROWS_PER_TILE)])
        plsc.subcore_barrier()

        @pl.loop(0, CHUNK_ITERS)
        def _(jj):
            j = jj * NS + s

            @pl.when(j < CHUNKS_PER_CORE)
            def _():
                off = c * EDGES_PER_CORE + j * CHUNK
                pltpu.sync_copy(src_hbm.at[pl.ds(off, CHUNK)], sidx)
                pltpu.sync_copy(dst_hbm.at[pl.ds(off, CHUNK)], didx)
                pltpu.sync_copy(g_hbm.at[sidx], rows)
                pltpu.sync_copy(rows, acc.at[didx], add=True)

        plsc.subcore_barrier()
        pltpu.sync_copy(acc.at[pl.ds(row0, ROWS_PER_TILE)],
                        out_hbm.at[c, pl.ds(row0, ROWS_PER_TILE)])

    return body(g, src, dst, zeros)


def _sc_degree(dst, zeros, ones_rows):
    @functools.partial(
        pl.kernel,
        out_type=jax.ShapeDtypeStruct((NC, N, D_HID), jnp.float32),
        mesh=_sc_mesh(),
        scratch_types=[
            pltpu.VMEM((CHUNK,), jnp.int32),
            pltpu.VMEM((CHUNK, D_HID), jnp.float32),
            pltpu.VMEM_SHARED((N, D_HID), jnp.float32),
        ],
    )
    def body(dst_hbm, z_hbm, ones_hbm, out_hbm, didx, rows, acc):
        c = lax.axis_index("c")
        s = lax.axis_index("s")
        row0 = s * ROWS_PER_TILE
        pltpu.sync_copy(ones_hbm, rows)
        pltpu.sync_copy(z_hbm.at[pl.ds(row0, ROWS_PER_TILE)],
                        acc.at[pl.ds(row0, ROWS_PER_TILE)])
        plsc.subcore_barrier()

        @pl.loop(0, CHUNK_ITERS)
        def _(jj):
            j = jj * NS + s

            @pl.when(j < CHUNKS_PER_CORE)
            def _():
                off = c * EDGES_PER_CORE + j * CHUNK
                pltpu.sync_copy(dst_hbm.at[pl.ds(off, CHUNK)], didx)
                pltpu.sync_copy(rows, acc.at[didx], add=True)

        plsc.subcore_barrier()
        pltpu.sync_copy(acc.at[pl.ds(row0, ROWS_PER_TILE)],
                        out_hbm.at[c, pl.ds(row0, ROWS_PER_TILE)])

    return body(dst, zeros, ones_rows)


def _tc_matmul1(x, w1):
    def body(x_ref, w_ref, o_ref):
        o_ref[...] = jnp.dot(x_ref[...], w_ref[...])

    return pl.pallas_call(
        body,
        grid=(NB,),
        in_specs=[
            pl.BlockSpec((BLK, D_IN), lambda i: (i, 0)),
            pl.BlockSpec((D_IN, D_HID), lambda i: (0, 0)),
        ],
        out_specs=pl.BlockSpec((BLK, D_HID), lambda i: (i, 0)),
        out_shape=jax.ShapeDtypeStruct((N, D_HID), jnp.float32),
    )(x, w1)


def _tc_norm_scale(deg_parts, h1):
    def body(dp_ref, h_ref, d_ref, g_ref):
        deg = dp_ref[0, :, 0] + dp_ref[1, :, 0] + 1.0
        d = lax.rsqrt(deg)[:, None]
        d_ref[...] = d
        g_ref[...] = h_ref[...] * d

    return pl.pallas_call(
        body,
        grid=(NB,),
        in_specs=[
            pl.BlockSpec((NC, BLK, D_HID), lambda i: (0, i, 0)),
            pl.BlockSpec((BLK, D_HID), lambda i: (i, 0)),
        ],
        out_specs=[
            pl.BlockSpec((BLK, 1), lambda i: (i, 0)),
            pl.BlockSpec((BLK, D_HID), lambda i: (i, 0)),
        ],
        out_shape=[
            jax.ShapeDtypeStruct((N, 1), jnp.float32),
            jax.ShapeDtypeStruct((N, D_HID), jnp.float32),
        ],
    )(deg_parts, h1)


def _tc_layer2(a1, g1, d, b1, w2):
    def body(a_ref, g_ref, d_ref, b_ref, w_ref, o_ref):
        d = d_ref[...]
        h = (a_ref[0] + a_ref[1] + g_ref[...]) * d + b_ref[...]
        h = jnp.maximum(h, 0.0)
        o_ref[...] = jnp.dot(h, w_ref[...]) * d

    return pl.pallas_call(
        body,
        grid=(NB,),
        in_specs=[
            pl.BlockSpec((NC, BLK, D_HID), lambda i: (0, i, 0)),
            pl.BlockSpec((BLK, D_HID), lambda i: (i, 0)),
            pl.BlockSpec((BLK, 1), lambda i: (i, 0)),
            pl.BlockSpec((1, D_HID), lambda i: (0, 0)),
            pl.BlockSpec((D_HID, D_HID), lambda i: (0, 0)),
        ],
        out_specs=pl.BlockSpec((BLK, D_HID), lambda i: (i, 0)),
        out_shape=jax.ShapeDtypeStruct((N, D_HID), jnp.float32),
    )(a1, g1, d, b1, w2)


def _tc_pool_head(a2, g2, d, b2, bi, w3, b3):
    def body(a_ref, g_ref, d_ref, bi_ref, b2_ref, w3_ref, b3_ref, o_ref,
             sums, cnts):
        i = pl.program_id(0)

        @pl.when(i == 0)
        def _():
            sums[...] = jnp.zeros_like(sums)
            cnts[...] = jnp.zeros_like(cnts)

        h = (a_ref[0] + a_ref[1] + g_ref[...]) * d_ref[...] + b2_ref[...]
        onehot = (bi_ref[...] == lax.broadcasted_iota(jnp.int32, (BLK, G), 1)
                  ).astype(jnp.float32)
        sums[...] += lax.dot_general(onehot, h, (((0,), (0,)), ((), ())))
        cnts[...] += lax.dot_general(
            onehot, jnp.ones((BLK, 1), jnp.float32), (((0,), (0,)), ((), ())))

        @pl.when(i == NB - 1)
        def _():
            pooled = sums[...] / jnp.maximum(cnts[...], 1.0)
            logits = jnp.dot(pooled, w3_ref[...]) + b3_ref[...]
            m = jnp.max(logits, axis=1, keepdims=True)
            e = jnp.exp(logits - m)
            o_ref[...] = e / jnp.sum(e, axis=1, keepdims=True)

    return pl.pallas_call(
        body,
        grid=(NB,),
        in_specs=[
            pl.BlockSpec((NC, BLK, D_HID), lambda i: (0, i, 0)),
            pl.BlockSpec((BLK, D_HID), lambda i: (i, 0)),
            pl.BlockSpec((BLK, 1), lambda i: (i, 0)),
            pl.BlockSpec((BLK, 1), lambda i: (i, 0)),
            pl.BlockSpec((1, D_HID), lambda i: (0, 0)),
            pl.BlockSpec((D_HID, D_OUT), lambda i: (0, 0)),
            pl.BlockSpec((1, D_OUT), lambda i: (0, 0)),
        ],
        out_specs=pl.BlockSpec((G, D_OUT), lambda i: (0, 0)),
        out_shape=jax.ShapeDtypeStruct((G, D_OUT), jnp.float32),
        scratch_shapes=[
            pltpu.VMEM((G, D_HID), jnp.float32),
            pltpu.VMEM((G, 1), jnp.float32),
        ],
    )(a2, g2, d, bi, b2, w3, b3)


def kernel(x, edge_index, batch_index, W1, b1, W2, b2, W3, b3):
    src = edge_index[0].astype(jnp.int32)
    dst = edge_index[1].astype(jnp.int32)
    bi = batch_index.astype(jnp.int32).reshape(N, 1)
    zeros = jnp.zeros((N, D_HID), jnp.float32)
    ones_rows = jnp.ones((CHUNK, D_HID), jnp.float32)

    deg_parts = _sc_degree(dst, zeros, ones_rows)
    h1 = _tc_matmul1(x, W1)
    d, g1 = _tc_norm_scale(deg_parts, h1)
    a1 = _sc_agg(g1, src, dst, zeros)
    g2 = _tc_layer2(a1, g1, d, b1.reshape(1, D_HID), W2)
    a2 = _sc_agg(g2, src, dst, zeros)
    return _tc_pool_head(a2, g2, d, b2.reshape(1, D_HID), bi,
                         W3, b3.reshape(1, D_OUT))


# speedup vs baseline: 20.4802x; 20.4802x over previous
import functools

import jax
import jax.numpy as jnp
from jax import lax
from jax.experimental import pallas as pl
from jax.experimental.pallas import tpu as pltpu
from jax.experimental.pallas import tpu_sc as plsc

N = 50000
E = 800000
G = 128
D_IN = 64
D_HID = 16
D_OUT = 10

NC = 2
NS = 16
CHUNK = 128
EDGES_PER_CORE = E // NC
CHUNKS_PER_CORE = EDGES_PER_CORE // CHUNK
CHUNK_ITERS = (CHUNKS_PER_CORE + NS - 1) // NS
N_PAD = 50048
ROWS_PER_TILE = N_PAD // NS

BLK = 2000
NB = N // BLK


def _sc_mesh():
    return plsc.VectorSubcoreMesh(core_axis_name="c", subcore_axis_name="s")


def _sc_agg(g, src, dst, zeros):
    @functools.partial(
        pl.kernel,
        out_type=jax.ShapeDtypeStruct((NC, N_PAD, D_HID), jnp.float32),
        mesh=_sc_mesh(),
        scratch_types=[
            pltpu.VMEM((CHUNK,), jnp.int32),
            pltpu.VMEM((CHUNK,), jnp.int32),
            pltpu.VMEM((CHUNK, D_HID), jnp.float32),
            pltpu.VMEM_SHARED((N_PAD, D_HID), jnp.float32),
        ],
        compiler_params=pltpu.CompilerParams(use_tc_tiling_on_sc=False),
    )
    def body(g_hbm, src_hbm, dst_hbm, z_hbm, out_hbm, sidx, didx, rows, acc):
        c = lax.axis_index("c")
        s = lax.axis_index("s")
        row0 = s * ROWS_PER_TILE
        pltpu.sync_copy(z_hbm.at[pl.ds(row0, ROWS_PER_TILE)],
                        acc.at[pl.ds(row0, ROWS_PER_TILE)])
        plsc.subcore_barrier()

        @pl.loop(0, CHUNK_ITERS)
        def _(jj):
            j = jj * NS + s

            @pl.when(j < CHUNKS_PER_CORE)
            def _():
                off = c * EDGES_PER_CORE + j * CHUNK
                pltpu.sync_copy(src_hbm.at[pl.ds(off, CHUNK)], sidx)
                pltpu.sync_copy(dst_hbm.at[pl.ds(off, CHUNK)], didx)
                pltpu.sync_copy(g_hbm.at[sidx], rows)
                pltpu.sync_copy(rows, acc.at[didx], add=True)

        plsc.subcore_barrier()
        pltpu.sync_copy(acc.at[pl.ds(row0, ROWS_PER_TILE)],
                        out_hbm.at[c, pl.ds(row0, ROWS_PER_TILE)])

    return body(g, src, dst, zeros)


def _sc_degree(dst, zeros, ones_rows):
    @functools.partial(
        pl.kernel,
        out_type=jax.ShapeDtypeStruct((NC, N_PAD, D_HID), jnp.float32),
        mesh=_sc_mesh(),
        scratch_types=[
            pltpu.VMEM((CHUNK,), jnp.int32),
            pltpu.VMEM((CHUNK, D_HID), jnp.float32),
            pltpu.VMEM_SHARED((N_PAD, D_HID), jnp.float32),
        ],
        compiler_params=pltpu.CompilerParams(use_tc_tiling_on_sc=False),
    )
    def body(dst_hbm, z_hbm, ones_hbm, out_hbm, didx, rows, acc):
        c = lax.axis_index("c")
        s = lax.axis_index("s")
        row0 = s * ROWS_PER_TILE
        pltpu.sync_copy(ones_hbm, rows)
        pltpu.sync_copy(z_hbm.at[pl.ds(row0, ROWS_PER_TILE)],
                        acc.at[pl.ds(row0, ROWS_PER_TILE)])
        plsc.subcore_barrier()

        @pl.loop(0, CHUNK_ITERS)
        def _(jj):
            j = jj * NS + s

            @pl.when(j < CHUNKS_PER_CORE)
            def _():
                off = c * EDGES_PER_CORE + j * CHUNK
                pltpu.sync_copy(dst_hbm.at[pl.ds(off, CHUNK)], didx)
                pltpu.sync_copy(rows, acc.at[didx], add=True)

        plsc.subcore_barrier()
        pltpu.sync_copy(acc.at[pl.ds(row0, ROWS_PER_TILE)],
                        out_hbm.at[c, pl.ds(row0, ROWS_PER_TILE)])

    return body(dst, zeros, ones_rows)


def _tc_matmul1(x, w1):
    def body(x_ref, w_ref, o_ref):
        o_ref[...] = jnp.dot(x_ref[...], w_ref[...])

    return pl.pallas_call(
        body,
        grid=(NB,),
        in_specs=[
            pl.BlockSpec((BLK, D_IN), lambda i: (i, 0)),
            pl.BlockSpec((D_IN, D_HID), lambda i: (0, 0)),
        ],
        out_specs=pl.BlockSpec((BLK, D_HID), lambda i: (i, 0)),
        out_shape=jax.ShapeDtypeStruct((N, D_HID), jnp.float32),
    )(x, w1)


def _tc_norm_scale(deg_parts, h1):
    def body(dp_ref, h_ref, d_ref, g_ref):
        deg = dp_ref[0, :, 0] + dp_ref[1, :, 0] + 1.0
        d = lax.rsqrt(deg)[:, None]
        d_ref[...] = d
        g_ref[...] = h_ref[...] * d

    return pl.pallas_call(
        body,
        grid=(NB,),
        in_specs=[
            pl.BlockSpec((NC, BLK, D_HID), lambda i: (0, i, 0)),
            pl.BlockSpec((BLK, D_HID), lambda i: (i, 0)),
        ],
        out_specs=[
            pl.BlockSpec((BLK, 1), lambda i: (i, 0)),
            pl.BlockSpec((BLK, D_HID), lambda i: (i, 0)),
        ],
        out_shape=[
            jax.ShapeDtypeStruct((N, 1), jnp.float32),
            jax.ShapeDtypeStruct((N, D_HID), jnp.float32),
        ],
    )(deg_parts, h1)


def _tc_layer2(a1, g1, d, b1, w2):
    def body(a_ref, g_ref, d_ref, b_ref, w_ref, o_ref):
        d = d_ref[...]
        h = (a_ref[0] + a_ref[1] + g_ref[...]) * d + b_ref[...]
        h = jnp.maximum(h, 0.0)
        o_ref[...] = jnp.dot(h, w_ref[...]) * d

    return pl.pallas_call(
        body,
        grid=(NB,),
        in_specs=[
            pl.BlockSpec((NC, BLK, D_HID), lambda i: (0, i, 0)),
            pl.BlockSpec((BLK, D_HID), lambda i: (i, 0)),
            pl.BlockSpec((BLK, 1), lambda i: (i, 0)),
            pl.BlockSpec((1, D_HID), lambda i: (0, 0)),
            pl.BlockSpec((D_HID, D_HID), lambda i: (0, 0)),
        ],
        out_specs=pl.BlockSpec((BLK, D_HID), lambda i: (i, 0)),
        out_shape=jax.ShapeDtypeStruct((N, D_HID), jnp.float32),
    )(a1, g1, d, b1, w2)


def _tc_pool_head(a2, g2, d, b2, bi, w3, b3):
    def body(a_ref, g_ref, d_ref, bi_ref, b2_ref, w3_ref, b3_ref, o_ref,
             sums, cnts):
        i = pl.program_id(0)

        @pl.when(i == 0)
        def _():
            sums[...] = jnp.zeros_like(sums)
            cnts[...] = jnp.zeros_like(cnts)

        h = (a_ref[0] + a_ref[1] + g_ref[...]) * d_ref[...] + b2_ref[...]
        onehot = (bi_ref[...] == lax.broadcasted_iota(jnp.int32, (BLK, G), 1)
                  ).astype(jnp.float32)
        sums[...] += lax.dot_general(onehot, h, (((0,), (0,)), ((), ())))
        cnts[...] += lax.dot_general(
            onehot, jnp.ones((BLK, 1), jnp.float32), (((0,), (0,)), ((), ())))

        @pl.when(i == NB - 1)
        def _():
            pooled = sums[...] / jnp.maximum(cnts[...], 1.0)
            logits = jnp.dot(pooled, w3_ref[...]) + b3_ref[...]
            m = jnp.max(logits, axis=1, keepdims=True)
            e = jnp.exp(logits - m)
            o_ref[...] = e / jnp.sum(e, axis=1, keepdims=True)

    return pl.pallas_call(
        body,
        grid=(NB,),
        in_specs=[
            pl.BlockSpec((NC, BLK, D_HID), lambda i: (0, i, 0)),
            pl.BlockSpec((BLK, D_HID), lambda i: (i, 0)),
            pl.BlockSpec((BLK, 1), lambda i: (i, 0)),
            pl.BlockSpec((BLK, 1), lambda i: (i, 0)),
            pl.BlockSpec((1, D_HID), lambda i: (0, 0)),
            pl.BlockSpec((D_HID, D_OUT), lambda i: (0, 0)),
            pl.BlockSpec((1, D_OUT), lambda i: (0, 0)),
        ],
        out_specs=pl.BlockSpec((G, D_OUT), lambda i: (0, 0)),
        out_shape=jax.ShapeDtypeStruct((G, D_OUT), jnp.float32),
        scratch_shapes=[
            pltpu.VMEM((G, D_HID), jnp.float32),
            pltpu.VMEM((G, 1), jnp.float32),
        ],
    )(a2, g2, d, bi, b2, w3, b3)


def kernel(x, edge_index, batch_index, W1, b1, W2, b2, W3, b3):
    src = edge_index[0].astype(jnp.int32)
    dst = edge_index[1].astype(jnp.int32)
    bi = batch_index.astype(jnp.int32).reshape(N, 1)
    zeros = jnp.zeros((N_PAD, D_HID), jnp.float32)
    ones_rows = jnp.ones((CHUNK, D_HID), jnp.float32)

    deg_parts = _sc_degree(dst, zeros, ones_rows)
    h1 = _tc_matmul1(x, W1)
    d, g1 = _tc_norm_scale(deg_parts, h1)
    a1 = _sc_agg(g1, src, dst, zeros)
    g2 = _tc_layer2(a1, g1, d, b1.reshape(1, D_HID), W2)
    a2 = _sc_agg(g2, src, dst, zeros)
    return _tc_pool_head(a2, g2, d, b2.reshape(1, D_HID), bi,
                         W3, b3.reshape(1, D_OUT))


# speedup vs baseline: 47.7201x; 2.3301x over previous
import functools

import jax
import jax.numpy as jnp
from jax import lax
from jax.experimental import pallas as pl
from jax.experimental.pallas import tpu as pltpu
from jax.experimental.pallas import tpu_sc as plsc

N = 50000
E = 800000
G = 128
D_IN = 64
D_HID = 16
D_OUT = 10

NC = 2
NS = 16
CHUNK = 128
EDGES_PER_CORE = E // NC
CHUNKS_PER_CORE = EDGES_PER_CORE // CHUNK
CHUNK_ITERS = (CHUNKS_PER_CORE + NS - 1) // NS
N_PAD = 50048
ROWS_PER_TILE = N_PAD // NS

BLK = 2000
NB = N // BLK


SS = 13
PAIRS = 7
CR_PER_CORE = CHUNKS_PER_CORE


def _sc_mesh():
    return plsc.VectorSubcoreMesh(core_axis_name="c", subcore_axis_name="s")


def _sc_agg(g, src2, dst2, zeros):
    @functools.partial(
        pl.kernel,
        out_type=jax.ShapeDtypeStruct((NC, N_PAD, D_HID), jnp.float32),
        mesh=_sc_mesh(),
        scratch_types=[
            pltpu.VMEM((SS, CHUNK), jnp.int32),
            pltpu.VMEM((SS, CHUNK), jnp.int32),
            pltpu.VMEM((SS, CHUNK), jnp.int32),
            pltpu.VMEM((SS, CHUNK), jnp.int32),
            pltpu.VMEM((SS, CHUNK, D_HID), jnp.float32),
            pltpu.VMEM((SS, CHUNK, D_HID), jnp.float32),
            pltpu.VMEM_SHARED((N_PAD, D_HID), jnp.float32),
            pltpu.SemaphoreType.DMA,
            pltpu.SemaphoreType.DMA,
        ],
        compiler_params=pltpu.CompilerParams(use_tc_tiling_on_sc=False),
    )
    def body(g_hbm, src_hbm, dst_hbm, z_hbm, out_hbm,
             sidx0, didx0, sidx1, didx1, rows0, rows1, acc, gsem, ssem):
        c = lax.axis_index("c")
        s = lax.axis_index("s")
        row0 = s * ROWS_PER_TILE
        pltpu.sync_copy(z_hbm.at[pl.ds(row0, ROWS_PER_TILE)],
                        acc.at[pl.ds(row0, ROWS_PER_TILE)])
        plsc.subcore_barrier()

        start = c * CR_PER_CORE + s * 195 + jnp.minimum(s, 5)

        def load_idx(r, sidx, didx):
            pltpu.sync_copy(src_hbm.at[pl.ds(r, SS)], sidx)
            pltpu.sync_copy(dst_hbm.at[pl.ds(r, SS)], didx)

        def fire_gathers(sidx, rows):
            return [pltpu.async_copy(g_hbm.at[sidx.at[u]], rows.at[u], gsem)
                    for u in range(SS)]

        def fire_scatters(didx, rows):
            return [pltpu.async_copy(rows.at[u], acc.at[didx.at[u]], ssem,
                                     add=True)
                    for u in range(SS)]

        def run_superstep_sync(r):
            load_idx(r, sidx0, didx0)
            hs = fire_gathers(sidx0, rows0)
            for h in hs:
                h.wait()
            sc = fire_scatters(didx0, rows0)
            for h in sc:
                h.wait()

        @pl.loop(0, PAIRS)
        def _(q):
            r = start + q * (2 * SS)
            load_idx(r, sidx0, didx0)
            ha = fire_gathers(sidx0, rows0)
            load_idx(r + SS, sidx1, didx1)
            for h in ha:
                h.wait()
            sa = fire_scatters(didx0, rows0)
            hb = fire_gathers(sidx1, rows1)
            for h in hb:
                h.wait()
            sb = fire_scatters(didx1, rows1)
            for h in sa + sb:
                h.wait()

        run_superstep_sync(start + PAIRS * 2 * SS)

        @pl.when(s < 5)
        def _():
            rt = c * CR_PER_CORE + 196 * s + 195
            pltpu.sync_copy(src_hbm.at[pl.ds(rt, 1)], sidx0.at[pl.ds(0, 1)])
            pltpu.sync_copy(dst_hbm.at[pl.ds(rt, 1)], didx0.at[pl.ds(0, 1)])
            pltpu.sync_copy(g_hbm.at[sidx0.at[0]], rows0.at[0])
            pltpu.sync_copy(rows0.at[0], acc.at[didx0.at[0]], add=True)

        plsc.subcore_barrier()
        pltpu.sync_copy(acc.at[pl.ds(row0, ROWS_PER_TILE)],
                        out_hbm.at[c, pl.ds(row0, ROWS_PER_TILE)])

    return body(g, src2, dst2, zeros)


def _sc_degree(dst2, zeros, ones_rows):
    @functools.partial(
        pl.kernel,
        out_type=jax.ShapeDtypeStruct((NC, N_PAD, D_HID), jnp.float32),
        mesh=_sc_mesh(),
        scratch_types=[
            pltpu.VMEM((SS, CHUNK), jnp.int32),
            pltpu.VMEM((SS, CHUNK), jnp.int32),
            pltpu.VMEM((CHUNK, D_HID), jnp.float32),
            pltpu.VMEM_SHARED((N_PAD, D_HID), jnp.float32),
            pltpu.SemaphoreType.DMA,
        ],
        compiler_params=pltpu.CompilerParams(use_tc_tiling_on_sc=False),
    )
    def body(dst_hbm, z_hbm, ones_hbm, out_hbm, didx0, didx1, obuf, acc, ssem):
        c = lax.axis_index("c")
        s = lax.axis_index("s")
        row0 = s * ROWS_PER_TILE
        pltpu.sync_copy(ones_hbm, obuf)
        pltpu.sync_copy(z_hbm.at[pl.ds(row0, ROWS_PER_TILE)],
                        acc.at[pl.ds(row0, ROWS_PER_TILE)])
        plsc.subcore_barrier()

        start = c * CR_PER_CORE + s * 195 + jnp.minimum(s, 5)

        def fire_scatters(didx):
            return [pltpu.async_copy(obuf, acc.at[didx.at[u]], ssem, add=True)
                    for u in range(SS)]

        @pl.loop(0, PAIRS)
        def _(q):
            r = start + q * (2 * SS)
            pltpu.sync_copy(dst_hbm.at[pl.ds(r, SS)], didx0)
            sa = fire_scatters(didx0)
            pltpu.sync_copy(dst_hbm.at[pl.ds(r + SS, SS)], didx1)
            sb = fire_scatters(didx1)
            for h in sa + sb:
                h.wait()

        r = start + PAIRS * 2 * SS
        pltpu.sync_copy(dst_hbm.at[pl.ds(r, SS)], didx0)
        sa = fire_scatters(didx0)
        for h in sa:
            h.wait()

        @pl.when(s < 5)
        def _():
            rt = c * CR_PER_CORE + 196 * s + 195
            pltpu.sync_copy(dst_hbm.at[pl.ds(rt, 1)], didx0.at[pl.ds(0, 1)])
            pltpu.sync_copy(obuf, acc.at[didx0.at[0]], add=True)

        plsc.subcore_barrier()
        pltpu.sync_copy(acc.at[pl.ds(row0, ROWS_PER_TILE)],
                        out_hbm.at[c, pl.ds(row0, ROWS_PER_TILE)])

    return body(dst2, zeros, ones_rows)


def _tc_matmul1(x, w1):
    def body(x_ref, w_ref, o_ref):
        o_ref[...] = jnp.dot(x_ref[...], w_ref[...])

    return pl.pallas_call(
        body,
        grid=(NB,),
        in_specs=[
            pl.BlockSpec((BLK, D_IN), lambda i: (i, 0)),
            pl.BlockSpec((D_IN, D_HID), lambda i: (0, 0)),
        ],
        out_specs=pl.BlockSpec((BLK, D_HID), lambda i: (i, 0)),
        out_shape=jax.ShapeDtypeStruct((N, D_HID), jnp.float32),
    )(x, w1)


def _tc_norm_scale(deg_parts, h1):
    def body(dp_ref, h_ref, d_ref, g_ref):
        deg = dp_ref[0, :, 0] + dp_ref[1, :, 0] + 1.0
        d = lax.rsqrt(deg)[:, None]
        d_ref[...] = d
        g_ref[...] = h_ref[...] * d

    return pl.pallas_call(
        body,
        grid=(NB,),
        in_specs=[
            pl.BlockSpec((NC, BLK, D_HID), lambda i: (0, i, 0)),
            pl.BlockSpec((BLK, D_HID), lambda i: (i, 0)),
        ],
        out_specs=[
            pl.BlockSpec((BLK, 1), lambda i: (i, 0)),
            pl.BlockSpec((BLK, D_HID), lambda i: (i, 0)),
        ],
        out_shape=[
            jax.ShapeDtypeStruct((N, 1), jnp.float32),
            jax.ShapeDtypeStruct((N, D_HID), jnp.float32),
        ],
    )(deg_parts, h1)


def _tc_layer2(a1, g1, d, b1, w2):
    def body(a_ref, g_ref, d_ref, b_ref, w_ref, o_ref):
        d = d_ref[...]
        h = (a_ref[0] + a_ref[1] + g_ref[...]) * d + b_ref[...]
        h = jnp.maximum(h, 0.0)
        o_ref[...] = jnp.dot(h, w_ref[...]) * d

    return pl.pallas_call(
        body,
        grid=(NB,),
        in_specs=[
            pl.BlockSpec((NC, BLK, D_HID), lambda i: (0, i, 0)),
            pl.BlockSpec((BLK, D_HID), lambda i: (i, 0)),
            pl.BlockSpec((BLK, 1), lambda i: (i, 0)),
            pl.BlockSpec((1, D_HID), lambda i: (0, 0)),
            pl.BlockSpec((D_HID, D_HID), lambda i: (0, 0)),
        ],
        out_specs=pl.BlockSpec((BLK, D_HID), lambda i: (i, 0)),
        out_shape=jax.ShapeDtypeStruct((N, D_HID), jnp.float32),
    )(a1, g1, d, b1, w2)


def _tc_pool_head(a2, g2, d, b2, bi, w3, b3):
    def body(a_ref, g_ref, d_ref, bi_ref, b2_ref, w3_ref, b3_ref, o_ref,
             sums, cnts):
        i = pl.program_id(0)

        @pl.when(i == 0)
        def _():
            sums[...] = jnp.zeros_like(sums)
            cnts[...] = jnp.zeros_like(cnts)

        h = (a_ref[0] + a_ref[1] + g_ref[...]) * d_ref[...] + b2_ref[...]
        onehot = (bi_ref[...] == lax.broadcasted_iota(jnp.int32, (BLK, G), 1)
                  ).astype(jnp.float32)
        sums[...] += lax.dot_general(onehot, h, (((0,), (0,)), ((), ())))
        cnts[...] += lax.dot_general(
            onehot, jnp.ones((BLK, 1), jnp.float32), (((0,), (0,)), ((), ())))

        @pl.when(i == NB - 1)
        def _():
            pooled = sums[...] / jnp.maximum(cnts[...], 1.0)
            logits = jnp.dot(pooled, w3_ref[...]) + b3_ref[...]
            m = jnp.max(logits, axis=1, keepdims=True)
            e = jnp.exp(logits - m)
            o_ref[...] = e / jnp.sum(e, axis=1, keepdims=True)

    return pl.pallas_call(
        body,
        grid=(NB,),
        in_specs=[
            pl.BlockSpec((NC, BLK, D_HID), lambda i: (0, i, 0)),
            pl.BlockSpec((BLK, D_HID), lambda i: (i, 0)),
            pl.BlockSpec((BLK, 1), lambda i: (i, 0)),
            pl.BlockSpec((BLK, 1), lambda i: (i, 0)),
            pl.BlockSpec((1, D_HID), lambda i: (0, 0)),
            pl.BlockSpec((D_HID, D_OUT), lambda i: (0, 0)),
            pl.BlockSpec((1, D_OUT), lambda i: (0, 0)),
        ],
        out_specs=pl.BlockSpec((G, D_OUT), lambda i: (0, 0)),
        out_shape=jax.ShapeDtypeStruct((G, D_OUT), jnp.float32),
        scratch_shapes=[
            pltpu.VMEM((G, D_HID), jnp.float32),
            pltpu.VMEM((G, 1), jnp.float32),
        ],
    )(a2, g2, d, bi, b2, w3, b3)


def kernel(x, edge_index, batch_index, W1, b1, W2, b2, W3, b3):
    src = edge_index[0].astype(jnp.int32).reshape(E // CHUNK, CHUNK)
    dst = edge_index[1].astype(jnp.int32).reshape(E // CHUNK, CHUNK)
    bi = batch_index.astype(jnp.int32).reshape(N, 1)
    zeros = jnp.zeros((N_PAD, D_HID), jnp.float32)
    ones_rows = jnp.ones((CHUNK, D_HID), jnp.float32)

    deg_parts = _sc_degree(dst, zeros, ones_rows)
    h1 = _tc_matmul1(x, W1)
    d, g1 = _tc_norm_scale(deg_parts, h1)
    a1 = _sc_agg(g1, src, dst, zeros)
    g2 = _tc_layer2(a1, g1, d, b1.reshape(1, D_HID), W2)
    a2 = _sc_agg(g2, src, dst, zeros)
    return _tc_pool_head(a2, g2, d, b2.reshape(1, D_HID), bi,
                         W3, b3.reshape(1, D_OUT))
